# baseline (speedup 1.0000x reference)
import jax
import jax.numpy as jnp
from jax.experimental import pallas as pl
from jax.experimental.pallas import tpu as pltpu

_N = 16384
_DIM = 2048
_E = 64
_TILE = 2048
_KCH = 1024
_NK = _DIM // _KCH


def _gate_tile(x_ref, w_ref, b_ref, vals_ref, idx_ref, acc_ref):
    k = pl.program_id(1)
    part = jax.lax.dot_general(
        x_ref[...], w_ref[...], (((1,), (1,)), ((), ())),
        preferred_element_type=jnp.float32)

    @pl.when(k == 0)
    def _init():
        acc_ref[...] = part

    @pl.when(k > 0)
    def _acc():
        acc_ref[...] += part

    @pl.when(k == _NK - 1)
    def _finish():
        logits = acc_ref[...] + b_ref[...]
        col = jax.lax.broadcasted_iota(jnp.int32, logits.shape, 1)

        m1 = jnp.max(logits, axis=1, keepdims=True)
        i1 = jnp.min(jnp.where(logits == m1, col, _E), axis=1, keepdims=True)

        masked = jnp.where(col == i1, -jnp.inf, logits)
        m2 = jnp.max(masked, axis=1, keepdims=True)
        i2 = jnp.min(jnp.where(masked == m2, col, _E), axis=1, keepdims=True)

        denom = jnp.sum(jnp.exp(logits - m1), axis=1, keepdims=True)
        v1 = 1.0 / denom
        v2 = jnp.exp(m2 - m1) * v1

        vals_ref[...] = jnp.concatenate([v1, v2], axis=1)
        idx_ref[...] = jnp.concatenate([i1, i2], axis=1)


def kernel(x, weight, bias):
    n = x.shape[0]
    grid = (n // _TILE, _NK)
    vals, idx = pl.pallas_call(
        _gate_tile,
        grid=grid,
        in_specs=[
            pl.BlockSpec((_TILE, _KCH), lambda i, k: (i, k)),
            pl.BlockSpec((_E, _KCH), lambda i, k: (0, k)),
            pl.BlockSpec((1, _E), lambda i, k: (0, 0)),
        ],
        out_specs=[
            pl.BlockSpec((_TILE, 2), lambda i, k: (i, 0)),
            pl.BlockSpec((_TILE, 2), lambda i, k: (i, 0)),
        ],
        out_shape=[
            jax.ShapeDtypeStruct((n, 2), jnp.float32),
            jax.ShapeDtypeStruct((n, 2), jnp.int32),
        ],
        scratch_shapes=[pltpu.VMEM((_TILE, _E), jnp.float32)],
        compiler_params=pltpu.CompilerParams(
            dimension_semantics=("arbitrary", "arbitrary")),
    )(x, weight, bias.reshape(1, _E))
    return vals, idx


# speedup vs baseline: 1.2477x; 1.2477x over previous
import jax
import jax.numpy as jnp
from jax.experimental import pallas as pl
from jax.experimental.pallas import tpu as pltpu

_N = 16384
_DIM = 2048
_E = 64
_TILE = 2048


def _gate_tile(x_ref, w_ref, b_ref, vals_ref, idx_ref):
    x = x_ref[...]
    w = w_ref[...]
    logits = jax.lax.dot_general(
        x, w, (((1,), (1,)), ((), ())), preferred_element_type=jnp.float32)
    logits = logits + b_ref[...]

    colf = jax.lax.broadcasted_iota(
        jnp.int32, logits.shape, 1).astype(jnp.float32)

    m1 = jnp.max(logits, axis=1, keepdims=True)
    i1f = jnp.min(jnp.where(logits == m1, colf, float(_E)),
                  axis=1, keepdims=True)

    masked = jnp.where(colf == i1f, -jnp.inf, logits)
    m2 = jnp.max(masked, axis=1, keepdims=True)
    i2f = jnp.min(jnp.where(masked == m2, colf, float(_E)),
                  axis=1, keepdims=True)

    denom = jnp.sum(jnp.exp(logits - m1), axis=1, keepdims=True)
    v1 = 1.0 / denom
    v2 = jnp.exp(m2 - m1) * v1

    vals_ref[...] = jnp.concatenate([v1, v2], axis=1)
    idx_ref[...] = jnp.concatenate([i1f, i2f], axis=1).astype(jnp.int32)


def kernel(x, weight, bias):
    n = x.shape[0]
    grid = (n // _TILE,)
    vals, idx = pl.pallas_call(
        _gate_tile,
        grid=grid,
        in_specs=[
            pl.BlockSpec((_TILE, _DIM), lambda i: (i, 0)),
            pl.BlockSpec((_E, _DIM), lambda i: (0, 0)),
            pl.BlockSpec((1, _E), lambda i: (0, 0)),
        ],
        out_specs=[
            pl.BlockSpec((_TILE, 2), lambda i: (i, 0)),
            pl.BlockSpec((_TILE, 2), lambda i: (i, 0)),
        ],
        out_shape=[
            jax.ShapeDtypeStruct((n, 2), jnp.float32),
            jax.ShapeDtypeStruct((n, 2), jnp.int32),
        ],
        compiler_params=pltpu.CompilerParams(
            dimension_semantics=("arbitrary",)),
    )(x, weight, bias.reshape(1, _E))
    return vals, idx
